# baseline (speedup 1.0000x reference)
import functools

import jax
import jax.numpy as jnp
from jax import lax
from jax.experimental import pallas as pl
from jax.experimental.pallas import tpu as pltpu
from jax.experimental.pallas import tpu_sc as plsc

D_DIST = 1.5
B, C, N, K = 16, 64, 8192, 32
NC, NS = 2, 16
NW = NC * NS
CH = C // 2
CHUNK = 1024
NCHUNK = N // CHUNK
NG = CHUNK // 16
LANES = 16


def _seg_body(pred, lab, sums_out, cnt_out, lab_v, buf, bins, cbins, sem0,
              sem1):
  s = lax.axis_index("s")
  c = lax.axis_index("c")
  wid = s * NC + c
  b = wid // 2
  chalf = wid % 2
  c0 = chalf * CH

  def copy_in(g, buf_i, sem):
    return pltpu.async_copy(
        pred.at[b, pl.ds(c0, CH), pl.ds(g * CHUNK, CHUNK)], buf.at[buf_i],
        sem)

  def wait_in(buf_i, sem):
    pltpu.make_async_copy(
        pred.at[b, pl.ds(c0, CH), pl.ds(0, CHUNK)], buf.at[buf_i],
        sem).wait()

  copy_in(0, 0, sem0)
  pltpu.sync_copy(lab.at[b], lab_v)

  zero16 = jnp.zeros((LANES,), jnp.float32)
  ones16 = jnp.ones((LANES,), jnp.float32)
  lane = lax.iota(jnp.int32, LANES)

  @plsc.parallel_loop(0, (CH * K * LANES) // LANES, unroll=8)
  def _(i):
    bins[pl.ds(i * LANES, LANES)] = zero16

  for i in range((K * LANES) // LANES):
    cbins[pl.ds(i * LANES, LANES)] = zero16

  @plsc.parallel_loop(0, N // LANES, unroll=4)
  def _(i):
    l16 = lab_v[pl.ds(i * LANES, LANES)] * LANES + lane
    lab_v[pl.ds(i * LANES, LANES)] = l16
    plsc.addupdate_scatter(cbins, [l16], ones16)

  def process(bufref, n0):
    @plsc.parallel_loop(0, NG, unroll=2)
    def _(i):
      idx16 = lab_v[pl.ds(n0 + i * LANES, LANES)]
      for ch in range(CH):
        v = bufref[ch, pl.ds(i * LANES, LANES)]
        plsc.addupdate_scatter(
            bins.at[pl.ds(ch * (K * LANES), K * LANES)], [idx16], v)

  nh = NCHUNK // 2

  def chunk_loop(h, _):
    g0 = 2 * h
    copy_in(g0 + 1, 1, sem1)
    wait_in(0, sem0)
    process(buf.at[0], g0 * CHUNK)

    @pl.when(h + 1 < nh)
    def _():
      copy_in(g0 + 2, 0, sem0)

    wait_in(1, sem1)
    process(buf.at[1], (g0 + 1) * CHUNK)
    return 0
  lax.fori_loop(0, nh, chunk_loop, 0)

  pltpu.sync_copy(bins, sums_out.at[b, chalf])

  @pl.when(chalf == 0)
  def _():
    pltpu.sync_copy(cbins, cnt_out.at[b, 0])


_seg_sums = functools.partial(
    pl.kernel,
    out_type=(
        jax.ShapeDtypeStruct((B, 2, CH * K * LANES), jnp.float32),
        jax.ShapeDtypeStruct((B, 1, K * LANES), jnp.float32),
    ),
    mesh=plsc.VectorSubcoreMesh(
        core_axis_name="c", subcore_axis_name="s", num_cores=NC,
        num_subcores=NS),
    compiler_params=pltpu.CompilerParams(needs_layout_passes=False),
    scratch_types=[
        pltpu.VMEM((N,), jnp.int32),
        pltpu.VMEM((2, CH, CHUNK), jnp.float32),
        pltpu.VMEM((CH * K * LANES,), jnp.float32),
        pltpu.VMEM((K * LANES,), jnp.float32),
        pltpu.SemaphoreType.DMA,
        pltpu.SemaphoreType.DMA,
    ],
)(_seg_body)


def _loss_body(sums_ref, cnt_ref, out_ref):
  q = lax.broadcasted_iota(jnp.int32, (K * LANES, K), 0)
  kk = lax.broadcasted_iota(jnp.int32, (K * LANES, K), 1)
  r_mat = (q // LANES == kk).astype(jnp.float32)
  loss = jnp.zeros((), jnp.float32)
  for b in range(B):
    x = sums_ref[b].reshape(C, K * LANES)
    ct = lax.dot_general(x, r_mat, (((1,), (0,)), ((), ())),
                         preferred_element_type=jnp.float32)
    cw = cnt_ref[b]
    cnt = lax.dot_general(cw, r_mat, (((1,), (0,)), ((), ())),
                          preferred_element_type=jnp.float32)[0]
    present = (cnt > 0.0).astype(jnp.float32)
    u = jnp.sum(present)
    safe = jnp.where(cnt > 0.0, cnt, 1.0)
    ct = ct / safe[None, :]
    gram = lax.dot_general(ct, ct, (((1,), (1,)), ((), ())),
                           preferred_element_type=jnp.float32)
    nrm = jnp.sum(ct * ct, axis=1)
    sq = nrm[:, None] + nrm[None, :] - 2.0 * gram
    d = jnp.sqrt(jax.nn.relu(sq))
    t = jax.nn.relu(2.0 * D_DIST - d)
    term = jnp.sum(t * t) / (2.0 * u * (u - 1.0 + 1e-16))
    loss = loss + jnp.where(u != 1.0, term, 0.0)
  out_ref[...] = loss.reshape(1, 1)


def kernel(prediction, label):
  sums, cnt = _seg_sums(prediction, label)
  loss = pl.pallas_call(
      _loss_body,
      out_shape=jax.ShapeDtypeStruct((1, 1), jnp.float32),
  )(sums, cnt)
  return loss[0, 0]


# speedup vs baseline: 1.0306x; 1.0306x over previous
import functools

import jax
import jax.numpy as jnp
from jax import lax
from jax.experimental import pallas as pl
from jax.experimental.pallas import tpu as pltpu
from jax.experimental.pallas import tpu_sc as plsc

D_DIST = 1.5
B, C, N, K = 16, 64, 8192, 32
NC, NS = 2, 16
NW = NC * NS
CH = C // 2
CHUNK = 1024
NCHUNK = N // CHUNK
NG = CHUNK // 16
LANES = 16


def _seg_body(pred, lab, sums_out, cnt_out, lab_v, buf, bins, cbins, sem0,
              sem1):
  s = lax.axis_index("s")
  c = lax.axis_index("c")
  wid = s * NC + c
  b = wid // 2
  chalf = wid % 2
  c0 = chalf * CH

  def copy_in(g, buf_i, sem):
    return pltpu.async_copy(
        pred.at[b, pl.ds(c0, CH), pl.ds(g * CHUNK, CHUNK)], buf.at[buf_i],
        sem)

  def wait_in(buf_i, sem):
    pltpu.make_async_copy(
        pred.at[b, pl.ds(c0, CH), pl.ds(0, CHUNK)], buf.at[buf_i],
        sem).wait()

  copy_in(0, 0, sem0)
  pltpu.sync_copy(lab.at[b], lab_v)

  zero16 = jnp.zeros((LANES,), jnp.float32)
  ones16 = jnp.ones((LANES,), jnp.float32)
  lane = lax.iota(jnp.int32, LANES)

  @plsc.parallel_loop(0, (CH * K * LANES) // LANES, unroll=8)
  def _(i):
    bins[pl.ds(i * LANES, LANES)] = zero16

  for i in range((K * LANES) // LANES):
    cbins[pl.ds(i * LANES, LANES)] = zero16

  @plsc.parallel_loop(0, N // LANES, unroll=4)
  def _(i):
    l16 = lab_v[pl.ds(i * LANES, LANES)] * LANES + lane
    lab_v[pl.ds(i * LANES, LANES)] = l16
    plsc.addupdate_scatter(cbins, [l16], ones16)

  def process(bufref, n0):
    @plsc.parallel_loop(0, NG, unroll=1)
    def _(i):
      idx16 = lab_v[pl.ds(n0 + i * LANES, LANES)]
      for ch in range(CH):
        v = bufref[ch, pl.ds(i * LANES, LANES)]
        plsc.addupdate_scatter(
            bins.at[pl.ds(ch * (K * LANES), K * LANES)], [idx16], v)

  nh = NCHUNK // 2

  def chunk_loop(h, _):
    g0 = 2 * h
    copy_in(g0 + 1, 1, sem1)
    wait_in(0, sem0)
    process(buf.at[0], g0 * CHUNK)

    @pl.when(h + 1 < nh)
    def _():
      copy_in(g0 + 2, 0, sem0)

    wait_in(1, sem1)
    process(buf.at[1], (g0 + 1) * CHUNK)
    return 0
  lax.fori_loop(0, nh, chunk_loop, 0)

  pltpu.sync_copy(bins, sums_out.at[b, chalf])

  @pl.when(chalf == 0)
  def _():
    pltpu.sync_copy(cbins, cnt_out.at[b, 0])


_seg_sums = functools.partial(
    pl.kernel,
    out_type=(
        jax.ShapeDtypeStruct((B, 2, CH * K * LANES), jnp.float32),
        jax.ShapeDtypeStruct((B, 1, K * LANES), jnp.float32),
    ),
    mesh=plsc.VectorSubcoreMesh(
        core_axis_name="c", subcore_axis_name="s", num_cores=NC,
        num_subcores=NS),
    compiler_params=pltpu.CompilerParams(needs_layout_passes=False),
    scratch_types=[
        pltpu.VMEM((N,), jnp.int32),
        pltpu.VMEM((2, CH, CHUNK), jnp.float32),
        pltpu.VMEM((CH * K * LANES,), jnp.float32),
        pltpu.VMEM((K * LANES,), jnp.float32),
        pltpu.SemaphoreType.DMA,
        pltpu.SemaphoreType.DMA,
    ],
)(_seg_body)


def _loss_body(sums_ref, cnt_ref, out_ref):
  q = lax.broadcasted_iota(jnp.int32, (K * LANES, K), 0)
  kk = lax.broadcasted_iota(jnp.int32, (K * LANES, K), 1)
  r_mat = (q // LANES == kk).astype(jnp.float32)
  loss = jnp.zeros((), jnp.float32)
  for b in range(B):
    x = sums_ref[b].reshape(C, K * LANES)
    ct = lax.dot_general(x, r_mat, (((1,), (0,)), ((), ())),
                         preferred_element_type=jnp.float32)
    cw = cnt_ref[b]
    cnt = lax.dot_general(cw, r_mat, (((1,), (0,)), ((), ())),
                          preferred_element_type=jnp.float32)[0]
    present = (cnt > 0.0).astype(jnp.float32)
    u = jnp.sum(present)
    safe = jnp.where(cnt > 0.0, cnt, 1.0)
    ct = ct / safe[None, :]
    gram = lax.dot_general(ct, ct, (((1,), (1,)), ((), ())),
                           preferred_element_type=jnp.float32)
    nrm = jnp.sum(ct * ct, axis=1)
    sq = nrm[:, None] + nrm[None, :] - 2.0 * gram
    d = jnp.sqrt(jax.nn.relu(sq))
    t = jax.nn.relu(2.0 * D_DIST - d)
    term = jnp.sum(t * t) / (2.0 * u * (u - 1.0 + 1e-16))
    loss = loss + jnp.where(u != 1.0, term, 0.0)
  out_ref[...] = loss.reshape(1, 1)


def kernel(prediction, label):
  sums, cnt = _seg_sums(prediction, label)
  loss = pl.pallas_call(
      _loss_body,
      out_shape=jax.ShapeDtypeStruct((1, 1), jnp.float32),
  )(sums, cnt)
  return loss[0, 0]
